# baseline (speedup 1.0000x reference)
import dataclasses
import functools

import jax
import jax.numpy as jnp
from jax import lax
from jax.experimental import pallas as pl
from jax.experimental.pallas import tpu as pltpu
from jax.experimental.pallas import tpu_sc as plsc

N = 31744
E = 524288
F_IN = 16
H = 8
C = 9
BS = 512
NB = 62
D_ATT = 640
NCLS = 3

N1 = 32768
CHUNK = 128
NCHUNKS = E // 16 // CHUNK
GROUPS = CHUNK // 16
ROW = 48
AROW = 40
BLKC = 8
NBLK = NCHUNKS // BLKC
NPT = N1 // 16
NROUND = NPT // CHUNK

_SLOPE = 0.2


def _elu(x):
    return jnp.where(x > 0, x, jnp.exp(jnp.minimum(x, 0.0)) - 1.0)


def _stats_body(x_ref, o_ref):
    @pl.when(pl.program_id(0) == 0)
    def _():
        o_ref[...] = jnp.zeros_like(o_ref)
    xv = x_ref[...]
    o_ref[...] += jnp.stack([jnp.sum(xv, axis=0), jnp.sum(xv * xv, axis=0)])


def _bn_stats(x_c):
    return pl.pallas_call(
        _stats_body,
        grid=(8,),
        in_specs=[pl.BlockSpec((N // 8, 80), lambda i: (i, 0))],
        out_specs=pl.BlockSpec((2, 80), lambda i: (0, 0)),
        out_shape=jax.ShapeDtypeStruct((2, 80), jnp.float32),
    )(x_c)


def _proj_body(x_ref, st_ref, bw_ref, bb_ref, wl_ref, bl_ref, wr_ref, br_ref,
               xl_ref, xr_ref):
    x = x_ref[0]
    s1 = st_ref[0, 0]
    s2 = st_ref[0, 1]
    mu = s1 / N
    var = s2 / N - mu * mu
    scale = bw_ref[0, 0] * lax.rsqrt(var + 1e-5)
    xh = (x - mu[None, :]) * scale[None, :] + bb_ref[0, 0][None, :]
    ml = jnp.dot(xh, wl_ref[0], preferred_element_type=jnp.float32) + bl_ref[0, 0][None, :]
    mr = jnp.dot(xh, wr_ref[0], preferred_element_type=jnp.float32) + br_ref[0, 0][None, :]
    xl_ref[0, 0] = ml[:, :ROW]
    xl_ref[0, 1] = ml[:, ROW:]
    xr_ref[0, 0] = mr[:, :ROW]
    xr_ref[0, 1] = mr[:, ROW:]


def _projections(x_p, stats_b, bn_w, bn_b, wl2, bl2, wr2, br2):
    bn_w, bn_b = bn_w[:, None, :], bn_b[:, None, :]
    bl2, br2 = bl2[:, None, :], br2[:, None, :]
    blk = 3968
    out_sh = jax.ShapeDtypeStruct((5, 2, N, ROW), jnp.float32)
    return pl.pallas_call(
        _proj_body,
        grid=(5, N // blk),
        in_specs=[
            pl.BlockSpec((1, blk, 16), lambda b, i: (b, i, 0)),
            pl.BlockSpec((1, 2, 16), lambda b, i: (b, 0, 0)),
            pl.BlockSpec((1, 1, 16), lambda b, i: (b, 0, 0)),
            pl.BlockSpec((1, 1, 16), lambda b, i: (b, 0, 0)),
            pl.BlockSpec((1, 16, 96), lambda b, i: (b, 0, 0)),
            pl.BlockSpec((1, 1, 96), lambda b, i: (b, 0, 0)),
            pl.BlockSpec((1, 16, 96), lambda b, i: (b, 0, 0)),
            pl.BlockSpec((1, 1, 96), lambda b, i: (b, 0, 0)),
        ],
        out_specs=[
            pl.BlockSpec((1, 2, blk, ROW), lambda b, i: (b, 0, i, 0)),
            pl.BlockSpec((1, 2, blk, ROW), lambda b, i: (b, 0, i, 0)),
        ],
        out_shape=[out_sh, out_sh],
    )(x_p, stats_b, bn_w, bn_b, wl2, bl2, wr2, br2)


def _sc_compiler_params():
    cp = pltpu.CompilerParams()
    fields = pltpu.CompilerParams.__dataclass_fields__
    if "needs_layout_passes" in fields:
        cp = dataclasses.replace(cp, needs_layout_passes=False)
    if "use_tc_tiling_on_sc" in fields:
        cp = dataclasses.replace(cp, use_tc_tiling_on_sc=False)
    return cp


def _edge_phase(xl_t, xr_t, edges, att_r):
    mesh = plsc.VectorSubcoreMesh(core_axis_name="c", subcore_axis_name="s")

    @functools.partial(
        pl.kernel,
        out_type=jax.ShapeDtypeStruct((5, 2, N, 16), jnp.float32),
        mesh=mesh,
        compiler_params=_sc_compiler_params(),
        scratch_types=[
            pltpu.VMEM((BLKC, CHUNK), jnp.int32),
            pltpu.VMEM((BLKC, CHUNK), jnp.int32),
            pltpu.VMEM((3, CHUNK, ROW), jnp.float32),
            pltpu.VMEM((3, CHUNK, ROW), jnp.float32),
            pltpu.VMEM((2, CHUNK, AROW), jnp.float32),
            pltpu.VMEM((64, 16), jnp.float32),
            pltpu.VMEM_SHARED((N, AROW), jnp.float32),
            pltpu.VMEM((384,), jnp.float32),
            pltpu.VMEM((40, 16), jnp.float32),
            pltpu.SemaphoreType.DMA,
            pltpu.SemaphoreType.DMA,
            pltpu.SemaphoreType.DMA,
            pltpu.SemaphoreType.DMA,
            pltpu.SemaphoreType.DMA,
        ],
    )
    def body(xl_hbm, xr_hbm, e_hbm, att_hbm, part_hbm,
             srcs, dsts, xlr2, xrr2, pay2, outp, acc, att_s, att_b,
             semg0, semg1, semg2, sems0, sems1):
        ci = lax.axis_index("c")
        si = lax.axis_index("s")
        pltpu.sync_copy(att_hbm, att_s)
        semg = [semg0, semg1, semg2]
        sems = [sems0, sems1]

        z16 = jnp.zeros((16,), jnp.float32)

        @pl.loop(0, 64)
        def _(r):
            outp[r, pl.ds(0, 16)] = z16

        @pl.loop(0, 5)
        def _(b):
            for h in range(4):
                for c in range(C):
                    sp = plsc.load_gather(
                        att_s, [jnp.full((16,), h * C + c, jnp.int32) + (b * 2 + ci) * 36])
                    att_b[h * C + c, pl.ds(0, 16)] = sp

            @pl.loop(0, NROUND)
            def _(k):
                base = jnp.minimum(si * NPT + k * CHUNK, N - CHUNK)

                if True:
                    stl = xlr2.at[0]
                    str_ = xrr2.at[0]
                    pay = pay2.at[0]
                    pltpu.sync_copy(xl_hbm.at[b, ci, pl.ds(base, CHUNK)], stl)
                    pltpu.sync_copy(xr_hbm.at[b, ci, pl.ds(base, CHUNK)], str_)

                    @pl.loop(0, GROUPS)
                    def _(g):
                        ni = lax.iota(jnp.int32, 16) + g * 16
                        for h in range(4):
                            alpha = None
                            xls = []
                            for c in range(C):
                                col = jnp.full((16,), h * 12 + c, jnp.int32)
                                xlv = plsc.load_gather(stl, [ni, col])
                                xrv = plsc.load_gather(str_, [ni, col])
                                t = xlv + xrv
                                lr = jnp.maximum(t, _SLOPE * t)
                                term = lr * att_b[h * C + c, pl.ds(0, 16)]
                                alpha = term if alpha is None else alpha + term
                                xls.append(xlv)
                            ea = jnp.exp(alpha)
                            for c in range(C):
                                colo = jnp.full((16,), h * 10 + c, jnp.int32)
                                plsc.store_scatter(pay, [ni, colo], ea * xls[c])
                            cold = jnp.full((16,), h * 10 + 9, jnp.int32)
                            plsc.store_scatter(pay, [ni, cold], ea)

                    pltpu.sync_copy(pay, acc.at[pl.ds(base, CHUNK)])

            plsc.subcore_barrier()

            def issue_gather(row, k):
                pltpu.async_copy(xl_hbm.at[b, ci].at[srcs.at[row]], xlr2.at[k], semg[k])
                pltpu.async_copy(xr_hbm.at[b, ci].at[dsts.at[row]], xrr2.at[k], semg[k])

            def wait_gather(row, k):
                pltpu.make_async_copy(xl_hbm.at[b, ci].at[srcs.at[row]], xlr2.at[k], semg[k]).wait()
                pltpu.make_async_copy(xr_hbm.at[b, ci].at[dsts.at[row]], xrr2.at[k], semg[k]).wait()

            def wait_scatter(row, k):
                pltpu.make_async_copy(pay2.at[k], acc.at[dsts.at[row]], sems[k]).wait()

            def compute(row, gk, pk):
                xlr = xlr2.at[gk]
                xrr = xrr2.at[gk]
                pay = pay2.at[pk]
                atv = [att_b[j, pl.ds(0, 16)] for j in range(4 * C)]

                @pl.loop(0, GROUPS)
                def _(g):
                    ei = lax.iota(jnp.int32, 16) + g * 16
                    for h in range(4):
                        alpha = None
                        xls = []
                        for c in range(C):
                            col = jnp.full((16,), h * 12 + c, jnp.int32)
                            xlv = plsc.load_gather(xlr, [ei, col])
                            xrv = plsc.load_gather(xrr, [ei, col])
                            t = xlv + xrv
                            lr = jnp.maximum(t, _SLOPE * t)
                            term = lr * atv[h * C + c]
                            alpha = term if alpha is None else alpha + term
                            xls.append(xlv)
                        ea = jnp.exp(alpha)
                        for c in range(C):
                            colo = jnp.full((16,), h * 10 + c, jnp.int32)
                            plsc.store_scatter(pay, [ei, colo], ea * xls[c])
                        cold = jnp.full((16,), h * 10 + 9, jnp.int32)
                        plsc.store_scatter(pay, [ei, cold], ea)

                pltpu.async_copy(pay, acc.at[dsts.at[row]], sems[pk], add=True)

            @pl.loop(0, NBLK)
            def _(blk):
                @pl.when(blk > 0)
                def _():
                    wait_scatter(BLKC - 2, 0)
                    wait_scatter(BLKC - 1, 1)

                pltpu.sync_copy(e_hbm.at[b, 0, si, pl.ds(blk * BLKC, BLKC)], srcs)
                pltpu.sync_copy(e_hbm.at[b, 1, si, pl.ds(blk * BLKC, BLKC)], dsts)
                issue_gather(0, 0)
                issue_gather(1, 1)
                issue_gather(2, 2)

                for j in range(BLKC):
                    gk = j % 3
                    pk = j % 2
                    wait_gather(j, gk)
                    if j >= 2:
                        wait_scatter(j - 2, pk)
                    compute(j, gk, pk)
                    if j + 3 < BLKC:
                        issue_gather(j + 3, gk)

            wait_scatter(BLKC - 2, 0)
            wait_scatter(BLKC - 1, 1)

            plsc.subcore_barrier()

            @pl.loop(0, NROUND)
            def _(k):
                nb = jnp.minimum(si * NPT + k * CHUNK, N - CHUNK)

                if True:
                    stage = pay2.at[0]
                    pltpu.sync_copy(acc.at[pl.ds(nb, CHUNK)], stage)

                    for half in range(2):
                        @pl.loop(0, 4)
                        def _(g):
                            lo = lax.iota(jnp.int32, 16) + g * 16
                            ni = lo + half * 64
                            recs = []
                            for h in range(4):
                                d = plsc.load_gather(stage, [ni, jnp.full((16,), h * 10 + 9, jnp.int32)])
                                recs.append(1.0 / d)
                            for c in range(C):
                                p = None
                                for h in range(4):
                                    v = plsc.load_gather(stage, [ni, jnp.full((16,), h * 10 + c, jnp.int32)])
                                    v = v * recs[h]
                                    p = v if p is None else p + v
                                plsc.store_scatter(outp, [lo, jnp.full((16,), c, jnp.int32)], p)

                        pltpu.sync_copy(outp, part_hbm.at[b, ci, pl.ds(nb + half * 64, 64)])

            plsc.subcore_barrier()

    return body(xl_t, xr_t, edges, att_r)


def _fuse_body(pv_ref, gb_ref, wq_ref, bq_ref, wk_ref, bk_ref, wv_ref, bv_ref,
               wf_ref, bf_ref, o_ref, qa, ka, va):
    b = pl.program_id(0)
    hb = _elu((pv_ref[0, 0] + pv_ref[0, 1]) * 0.125 + gb_ref[0, 0][None, :])

    @pl.when(b == 0)
    def _():
        qa[...] = jnp.broadcast_to(bq_ref[...], (BS, D_ATT))
        ka[...] = jnp.broadcast_to(bk_ref[...], (BS, D_ATT))
        va[...] = jnp.broadcast_to(bv_ref[...], (BS, D_ATT))

    qa[...] += jnp.dot(hb, wq_ref[0], preferred_element_type=jnp.float32)
    ka[...] += jnp.dot(hb, wk_ref[0], preferred_element_type=jnp.float32)
    va[...] += jnp.dot(hb, wv_ref[0], preferred_element_type=jnp.float32)

    @pl.when(b == 4)
    def _():
        s = lax.dot_general(qa[...], ka[...], (((1,), (1,)), ((), ())),
                            preferred_element_type=jnp.float32)
        s = s * (1.0 / (D_ATT ** 0.5))
        s = s - jnp.max(s, axis=1, keepdims=True)
        p = jnp.exp(s)
        p = p / jnp.sum(p, axis=1, keepdims=True)
        o = jnp.dot(p, va[...], preferred_element_type=jnp.float32)
        o = jnp.dot(o, wf_ref[...], preferred_element_type=jnp.float32) + bf_ref[...][None, :]
        o_ref[...] = _elu(o)


def _fusion(pv, gbp, wqp, bq, wkp, bk, wvp, bv, wfp, bfp):
    dpad = NB * 16
    gbp = gbp[:, None, :]
    return pl.pallas_call(
        _fuse_body,
        grid=(5,),
        in_specs=[
            pl.BlockSpec((1, 2, BS, dpad), lambda b: (b, 0, 0, 0)),
            pl.BlockSpec((1, 1, dpad), lambda b: (b, 0, 0)),
            pl.BlockSpec((1, dpad, D_ATT), lambda b: (b, 0, 0)),
            pl.BlockSpec((D_ATT,), lambda b: (0,)),
            pl.BlockSpec((1, dpad, D_ATT), lambda b: (b, 0, 0)),
            pl.BlockSpec((D_ATT,), lambda b: (0,)),
            pl.BlockSpec((1, dpad, D_ATT), lambda b: (b, 0, 0)),
            pl.BlockSpec((D_ATT,), lambda b: (0,)),
            pl.BlockSpec((D_ATT, 128), lambda b: (0, 0)),
            pl.BlockSpec((128,), lambda b: (0,)),
        ],
        out_specs=pl.BlockSpec((BS, 128), lambda b: (0, 0)),
        out_shape=jax.ShapeDtypeStruct((BS, 128), jnp.float32),
        scratch_shapes=[
            pltpu.VMEM((BS, D_ATT), jnp.float32),
            pltpu.VMEM((BS, D_ATT), jnp.float32),
            pltpu.VMEM((BS, D_ATT), jnp.float32),
        ],
    )(pv, gbp, wqp, bq, wkp, bk, wvp, bv, wfp, bfp)


def _pack_proj_w(w):
    w4 = w.reshape(5, F_IN, 2, 4, C)
    w4 = jnp.pad(w4, ((0, 0), (0, 0), (0, 0), (0, 0), (0, 3)))
    return w4.reshape(5, F_IN, 96)


def _pack_proj_b(bv):
    b4 = bv.reshape(5, 2, 4, C)
    b4 = jnp.pad(b4, ((0, 0), (0, 0), (0, 0), (0, 3)))
    return b4.reshape(5, 96)


def _pack_fuse_w(w):
    w4 = w.reshape(NB, 5, C, D_ATT)
    w4 = jnp.pad(w4, ((0, 0), (0, 0), (0, 7), (0, 0)))
    return jnp.transpose(w4, (1, 0, 2, 3)).reshape(5, NB * 16, D_ATT)


def kernel(delta_x, alpha_x, beta_x, theta_x, gamma_x, batch,
           delta_edge_index, alpha_edge_index, beta_edge_index,
           theta_edge_index, gamma_edge_index,
           bn_w, bn_b, Wl, bl, Wr, br, att, gat_bias,
           Wq, bq, Wk, bk, Wv, bv, Wf, bf):
    xs = [delta_x, alpha_x, beta_x, theta_x, gamma_x]
    eis = [delta_edge_index, alpha_edge_index, beta_edge_index,
           theta_edge_index, gamma_edge_index]

    x_c = jnp.concatenate(xs, axis=1)
    x_p = jnp.stack(xs)

    edges = jnp.stack(eis).reshape(5, 2, 16, NCHUNKS, CHUNK)

    wl2, wr2 = _pack_proj_w(Wl), _pack_proj_w(Wr)
    bl2, br2 = _pack_proj_b(bl), _pack_proj_b(br)
    att_r = jnp.pad(att.reshape(-1), (0, 384 - 5 * 2 * 4 * C))

    gbp = jnp.tile(jnp.pad(gat_bias, ((0, 0), (0, 7)))[:, None, :], (1, NB, 1))
    gbp = gbp.reshape(5, NB * 16)
    wqp, wkp, wvp = _pack_fuse_w(Wq), _pack_fuse_w(Wk), _pack_fuse_w(Wv)
    wfp = jnp.pad(Wf, ((0, 0), (0, 128 - NCLS)))
    bfp = jnp.pad(bf, ((0, 128 - NCLS),))

    stats = _bn_stats(x_c)
    stats_b = jnp.transpose(stats.reshape(2, 5, 16), (1, 0, 2))

    xl_t, xr_t = _projections(x_p, stats_b, bn_w, bn_b, wl2, bl2, wr2, br2)

    part = _edge_phase(xl_t, xr_t, edges, att_r)

    pv = part.reshape(5, 2, BS, NB * 16)
    o = _fusion(pv, gbp, wqp, bq, wkp, bk, wvp, bv, wfp, bfp)
    return o[:, :NCLS]


# speedup vs baseline: 1.3798x; 1.3798x over previous
import dataclasses
import functools

import jax
import jax.numpy as jnp
from jax import lax
from jax.experimental import pallas as pl
from jax.experimental.pallas import tpu as pltpu
from jax.experimental.pallas import tpu_sc as plsc

N = 31744
E = 524288
F_IN = 16
H = 8
C = 9
BS = 512
NB = 62
D_ATT = 640
NCLS = 3

N1 = 32768
CHUNK = 128
NCHUNKS = E // 16 // CHUNK
GROUPS = CHUNK // 16
ROW = 40
AROW = 40
BLKC = 8
NBLK = NCHUNKS // BLKC
NPT = N1 // 16
NROUND = NPT // CHUNK

_SLOPE = 0.2


def _elu(x):
    return jnp.where(x > 0, x, jnp.exp(jnp.minimum(x, 0.0)) - 1.0)


def _stats_body(x_ref, o_ref):
    @pl.when(pl.program_id(0) == 0)
    def _():
        o_ref[...] = jnp.zeros_like(o_ref)
    xv = x_ref[...]
    o_ref[...] += jnp.stack([jnp.sum(xv, axis=0), jnp.sum(xv * xv, axis=0)])


def _bn_stats(x_c):
    return pl.pallas_call(
        _stats_body,
        grid=(8,),
        in_specs=[pl.BlockSpec((N // 8, 80), lambda i: (i, 0))],
        out_specs=pl.BlockSpec((2, 80), lambda i: (0, 0)),
        out_shape=jax.ShapeDtypeStruct((2, 80), jnp.float32),
    )(x_c)


def _proj_body(x_ref, st_ref, bw_ref, bb_ref, wl_ref, bl_ref, wr_ref, br_ref,
               xl_ref, xr_ref):
    x = x_ref[0]
    s1 = st_ref[0, 0]
    s2 = st_ref[0, 1]
    mu = s1 / N
    var = s2 / N - mu * mu
    scale = bw_ref[0, 0] * lax.rsqrt(var + 1e-5)
    xh = (x - mu[None, :]) * scale[None, :] + bb_ref[0, 0][None, :]
    ml = jnp.dot(xh, wl_ref[0], preferred_element_type=jnp.float32) + bl_ref[0, 0][None, :]
    mr = jnp.dot(xh, wr_ref[0], preferred_element_type=jnp.float32) + br_ref[0, 0][None, :]
    xl_ref[0, 0] = ml[:, :ROW]
    xl_ref[0, 1] = ml[:, ROW:2 * ROW]
    xr_ref[0, 0] = mr[:, :ROW]
    xr_ref[0, 1] = mr[:, ROW:2 * ROW]


def _projections(x_p, stats_b, bn_w, bn_b, wl2, bl2, wr2, br2):
    bn_w, bn_b = bn_w[:, None, :], bn_b[:, None, :]
    bl2, br2 = bl2[:, None, :], br2[:, None, :]
    blk = 3968
    out_sh = jax.ShapeDtypeStruct((5, 2, N, ROW), jnp.float32)
    return pl.pallas_call(
        _proj_body,
        grid=(5, N // blk),
        in_specs=[
            pl.BlockSpec((1, blk, 16), lambda b, i: (b, i, 0)),
            pl.BlockSpec((1, 2, 16), lambda b, i: (b, 0, 0)),
            pl.BlockSpec((1, 1, 16), lambda b, i: (b, 0, 0)),
            pl.BlockSpec((1, 1, 16), lambda b, i: (b, 0, 0)),
            pl.BlockSpec((1, 16, 80), lambda b, i: (b, 0, 0)),
            pl.BlockSpec((1, 1, 80), lambda b, i: (b, 0, 0)),
            pl.BlockSpec((1, 16, 80), lambda b, i: (b, 0, 0)),
            pl.BlockSpec((1, 1, 80), lambda b, i: (b, 0, 0)),
        ],
        out_specs=[
            pl.BlockSpec((1, 2, blk, ROW), lambda b, i: (b, 0, i, 0)),
            pl.BlockSpec((1, 2, blk, ROW), lambda b, i: (b, 0, i, 0)),
        ],
        out_shape=[out_sh, out_sh],
    )(x_p, stats_b, bn_w, bn_b, wl2, bl2, wr2, br2)


def _sc_compiler_params():
    cp = pltpu.CompilerParams()
    fields = pltpu.CompilerParams.__dataclass_fields__
    if "needs_layout_passes" in fields:
        cp = dataclasses.replace(cp, needs_layout_passes=False)
    if "use_tc_tiling_on_sc" in fields:
        cp = dataclasses.replace(cp, use_tc_tiling_on_sc=False)
    return cp


def _edge_phase(xl_t, xr_t, edges, att_r):
    mesh = plsc.VectorSubcoreMesh(core_axis_name="c", subcore_axis_name="s")

    @functools.partial(
        pl.kernel,
        out_type=jax.ShapeDtypeStruct((5, 2, N, 16), jnp.float32),
        mesh=mesh,
        compiler_params=_sc_compiler_params(),
        scratch_types=[
            pltpu.VMEM((2, BLKC, CHUNK), jnp.int32),
            pltpu.VMEM((2, BLKC, CHUNK), jnp.int32),
            pltpu.VMEM((2, CHUNK, ROW), jnp.float32),
            pltpu.VMEM((2, CHUNK, ROW), jnp.float32),
            pltpu.VMEM((2, CHUNK, AROW), jnp.float32),
            pltpu.VMEM((64, 16), jnp.float32),
            pltpu.VMEM_SHARED((N, AROW), jnp.float32),
            pltpu.VMEM((384,), jnp.float32),
            pltpu.VMEM((40, 16), jnp.float32),
            pltpu.SemaphoreType.DMA,
            pltpu.SemaphoreType.DMA,
            pltpu.SemaphoreType.DMA,
            pltpu.SemaphoreType.DMA,
            pltpu.SemaphoreType.DMA,
            pltpu.SemaphoreType.DMA,
        ],
    )
    def body(xl_hbm, xr_hbm, e_hbm, att_hbm, part_hbm,
             srcs2, dsts2, xlr2, xrr2, pay2, outp, acc, att_s, att_b,
             semg0, semg1, sems0, sems1, semi0, semi1):
        ci = lax.axis_index("c")
        si = lax.axis_index("s")
        pltpu.sync_copy(att_hbm, att_s)
        semg = [semg0, semg1]
        sems = [sems0, sems1]
        semi = [semi0, semi1]

        z16 = jnp.zeros((16,), jnp.float32)

        @pl.loop(0, 64)
        def _(r):
            outp[r, pl.ds(0, 16)] = z16

        @pl.loop(0, 5)
        def _(b):
            for h in range(4):
                for c in range(C):
                    sp = plsc.load_gather(
                        att_s, [jnp.full((16,), h * C + c, jnp.int32) + (b * 2 + ci) * 36])
                    att_b[h * C + c, pl.ds(0, 16)] = sp

            @pl.loop(0, NROUND)
            def _(k):
                base = jnp.minimum(si * NPT + k * CHUNK, N - CHUNK)

                if True:
                    stl = xlr2.at[0]
                    str_ = xrr2.at[0]
                    pay = pay2.at[0]
                    pltpu.sync_copy(xl_hbm.at[b, ci, pl.ds(base, CHUNK)], stl)
                    pltpu.sync_copy(xr_hbm.at[b, ci, pl.ds(base, CHUNK)], str_)

                    @pl.loop(0, GROUPS)
                    def _(g):
                        ni = lax.iota(jnp.int32, 16) + g * 16
                        for h in range(4):
                            alpha = None
                            xls = []
                            for c in range(C):
                                col = jnp.full((16,), h * 10 + c, jnp.int32)
                                xlv = plsc.load_gather(stl, [ni, col])
                                xrv = plsc.load_gather(str_, [ni, col])
                                t = xlv + xrv
                                lr = jnp.maximum(t, _SLOPE * t)
                                term = lr * att_b[h * C + c, pl.ds(0, 16)]
                                alpha = term if alpha is None else alpha + term
                                xls.append(xlv)
                            ea = jnp.exp(alpha)
                            for c in range(C):
                                colo = jnp.full((16,), h * 10 + c, jnp.int32)
                                plsc.store_scatter(pay, [ni, colo], ea * xls[c])
                            cold = jnp.full((16,), h * 10 + 9, jnp.int32)
                            plsc.store_scatter(pay, [ni, cold], ea)

                    pltpu.sync_copy(pay, acc.at[pl.ds(base, CHUNK)])

            plsc.subcore_barrier()

            def issue_gather(ib, row, k):
                pltpu.async_copy(xl_hbm.at[b, ci].at[srcs2.at[ib, row]], xlr2.at[k], semg[k])
                pltpu.async_copy(xr_hbm.at[b, ci].at[dsts2.at[ib, row]], xrr2.at[k], semg[k])

            def wait_gather(ib, row, k):
                pltpu.make_async_copy(xl_hbm.at[b, ci].at[srcs2.at[ib, row]], xlr2.at[k], semg[k]).wait()
                pltpu.make_async_copy(xr_hbm.at[b, ci].at[dsts2.at[ib, row]], xrr2.at[k], semg[k]).wait()

            def wait_scatter(ib, row, k):
                pltpu.make_async_copy(pay2.at[k], acc.at[dsts2.at[ib, row]], sems[k]).wait()

            def issue_idx(ib, blkq):
                pltpu.async_copy(e_hbm.at[b, 0, si, pl.ds(blkq * BLKC, BLKC)], srcs2.at[ib], semi[ib])
                pltpu.async_copy(e_hbm.at[b, 1, si, pl.ds(blkq * BLKC, BLKC)], dsts2.at[ib], semi[ib])

            def wait_idx(ib):
                pltpu.make_async_copy(e_hbm.at[b, 0, si, pl.ds(0, BLKC)], srcs2.at[ib], semi[ib]).wait()
                pltpu.make_async_copy(e_hbm.at[b, 1, si, pl.ds(0, BLKC)], dsts2.at[ib], semi[ib]).wait()

            def compute(ib, row, gk, pk):
                xlr = xlr2.at[gk]
                xrr = xrr2.at[gk]
                pay = pay2.at[pk]
                atv = [att_b[j, pl.ds(0, 16)] for j in range(4 * C)]

                @pl.loop(0, GROUPS)
                def _(g):
                    ei = lax.iota(jnp.int32, 16) + g * 16
                    for h in range(4):
                        alpha = None
                        xls = []
                        for c in range(C):
                            col = jnp.full((16,), h * 10 + c, jnp.int32)
                            xlv = plsc.load_gather(xlr, [ei, col])
                            xrv = plsc.load_gather(xrr, [ei, col])
                            t = xlv + xrv
                            lr = jnp.maximum(t, _SLOPE * t)
                            term = lr * atv[h * C + c]
                            alpha = term if alpha is None else alpha + term
                            xls.append(xlv)
                        ea = jnp.exp(alpha)
                        for c in range(C):
                            colo = jnp.full((16,), h * 10 + c, jnp.int32)
                            plsc.store_scatter(pay, [ei, colo], ea * xls[c])
                        cold = jnp.full((16,), h * 10 + 9, jnp.int32)
                        plsc.store_scatter(pay, [ei, cold], ea)

                pltpu.async_copy(pay, acc.at[dsts2.at[ib, row]], sems[pk], add=True)

            issue_idx(0, 0)

            @pl.loop(0, NBLK, step=2)
            def _(blk):
                for sub in range(2):
                    bb = blk + sub
                    ib = sub

                    @pl.when(bb > 0)
                    def _():
                        wait_scatter(1 - ib, BLKC - 2, 0)
                        wait_scatter(1 - ib, BLKC - 1, 1)

                    nxt = jnp.minimum(bb + 1, NBLK - 1)
                    issue_idx(1 - ib, nxt)

                    wait_idx(ib)
                    issue_gather(ib, 0, 0)
                    issue_gather(ib, 1, 1)

                    for j in range(BLKC):
                        k = j % 2
                        wait_gather(ib, j, k)
                        if j >= 2:
                            wait_scatter(ib, j - 2, k)
                        compute(ib, j, k, k)
                        if j + 2 < BLKC:
                            issue_gather(ib, j + 2, k)

            wait_scatter(1, BLKC - 2, 0)
            wait_scatter(1, BLKC - 1, 1)
            wait_idx(0)

            plsc.subcore_barrier()

            @pl.loop(0, NROUND)
            def _(k):
                nb = jnp.minimum(si * NPT + k * CHUNK, N - CHUNK)

                if True:
                    stage = pay2.at[0]
                    pltpu.sync_copy(acc.at[pl.ds(nb, CHUNK)], stage)

                    for half in range(2):
                        @pl.loop(0, 4)
                        def _(g):
                            lo = lax.iota(jnp.int32, 16) + g * 16
                            ni = lo + half * 64
                            recs = []
                            for h in range(4):
                                d = plsc.load_gather(stage, [ni, jnp.full((16,), h * 10 + 9, jnp.int32)])
                                recs.append(1.0 / d)
                            for c in range(C):
                                p = None
                                for h in range(4):
                                    v = plsc.load_gather(stage, [ni, jnp.full((16,), h * 10 + c, jnp.int32)])
                                    v = v * recs[h]
                                    p = v if p is None else p + v
                                plsc.store_scatter(outp, [lo, jnp.full((16,), c, jnp.int32)], p)

                        pltpu.sync_copy(outp, part_hbm.at[b, ci, pl.ds(nb + half * 64, 64)])

            plsc.subcore_barrier()

    return body(xl_t, xr_t, edges, att_r)


def _fuse_body(pv_ref, gb_ref, wq_ref, bq_ref, wk_ref, bk_ref, wv_ref, bv_ref,
               wf_ref, bf_ref, o_ref, qa, ka, va):
    b = pl.program_id(0)
    hb = _elu((pv_ref[0, 0] + pv_ref[0, 1]) * 0.125 + gb_ref[0, 0][None, :])

    @pl.when(b == 0)
    def _():
        qa[...] = jnp.broadcast_to(bq_ref[...], (BS, D_ATT))
        ka[...] = jnp.broadcast_to(bk_ref[...], (BS, D_ATT))
        va[...] = jnp.broadcast_to(bv_ref[...], (BS, D_ATT))

    qa[...] += jnp.dot(hb, wq_ref[0], preferred_element_type=jnp.float32)
    ka[...] += jnp.dot(hb, wk_ref[0], preferred_element_type=jnp.float32)
    va[...] += jnp.dot(hb, wv_ref[0], preferred_element_type=jnp.float32)

    @pl.when(b == 4)
    def _():
        s = lax.dot_general(qa[...], ka[...], (((1,), (1,)), ((), ())),
                            preferred_element_type=jnp.float32)
        s = s * (1.0 / (D_ATT ** 0.5))
        s = s - jnp.max(s, axis=1, keepdims=True)
        p = jnp.exp(s)
        p = p / jnp.sum(p, axis=1, keepdims=True)
        o = jnp.dot(p, va[...], preferred_element_type=jnp.float32)
        o = jnp.dot(o, wf_ref[...], preferred_element_type=jnp.float32) + bf_ref[...][None, :]
        o_ref[...] = _elu(o)


def _fusion(pv, gbp, wqp, bq, wkp, bk, wvp, bv, wfp, bfp):
    dpad = NB * 16
    gbp = gbp[:, None, :]
    return pl.pallas_call(
        _fuse_body,
        grid=(5,),
        in_specs=[
            pl.BlockSpec((1, 2, BS, dpad), lambda b: (b, 0, 0, 0)),
            pl.BlockSpec((1, 1, dpad), lambda b: (b, 0, 0)),
            pl.BlockSpec((1, dpad, D_ATT), lambda b: (b, 0, 0)),
            pl.BlockSpec((D_ATT,), lambda b: (0,)),
            pl.BlockSpec((1, dpad, D_ATT), lambda b: (b, 0, 0)),
            pl.BlockSpec((D_ATT,), lambda b: (0,)),
            pl.BlockSpec((1, dpad, D_ATT), lambda b: (b, 0, 0)),
            pl.BlockSpec((D_ATT,), lambda b: (0,)),
            pl.BlockSpec((D_ATT, 128), lambda b: (0, 0)),
            pl.BlockSpec((128,), lambda b: (0,)),
        ],
        out_specs=pl.BlockSpec((BS, 128), lambda b: (0, 0)),
        out_shape=jax.ShapeDtypeStruct((BS, 128), jnp.float32),
        scratch_shapes=[
            pltpu.VMEM((BS, D_ATT), jnp.float32),
            pltpu.VMEM((BS, D_ATT), jnp.float32),
            pltpu.VMEM((BS, D_ATT), jnp.float32),
        ],
    )(pv, gbp, wqp, bq, wkp, bk, wvp, bv, wfp, bfp)


def _pack_proj_w(w):
    w4 = w.reshape(5, F_IN, 2, 4, C)
    w4 = jnp.pad(w4, ((0, 0), (0, 0), (0, 0), (0, 0), (0, 1)))
    return w4.reshape(5, F_IN, 80)


def _pack_proj_b(bv):
    b4 = bv.reshape(5, 2, 4, C)
    b4 = jnp.pad(b4, ((0, 0), (0, 0), (0, 0), (0, 1)))
    return b4.reshape(5, 80)


def _pack_fuse_w(w):
    w4 = w.reshape(NB, 5, C, D_ATT)
    w4 = jnp.pad(w4, ((0, 0), (0, 0), (0, 7), (0, 0)))
    return jnp.transpose(w4, (1, 0, 2, 3)).reshape(5, NB * 16, D_ATT)


def kernel(delta_x, alpha_x, beta_x, theta_x, gamma_x, batch,
           delta_edge_index, alpha_edge_index, beta_edge_index,
           theta_edge_index, gamma_edge_index,
           bn_w, bn_b, Wl, bl, Wr, br, att, gat_bias,
           Wq, bq, Wk, bk, Wv, bv, Wf, bf):
    xs = [delta_x, alpha_x, beta_x, theta_x, gamma_x]
    eis = [delta_edge_index, alpha_edge_index, beta_edge_index,
           theta_edge_index, gamma_edge_index]

    x_c = jnp.concatenate(xs, axis=1)
    x_p = jnp.stack(xs)

    edges = jnp.stack(eis).reshape(5, 2, 16, NCHUNKS, CHUNK)

    wl2, wr2 = _pack_proj_w(Wl), _pack_proj_w(Wr)
    bl2, br2 = _pack_proj_b(bl), _pack_proj_b(br)
    att_r = jnp.pad(att.reshape(-1), (0, 384 - 5 * 2 * 4 * C))

    gbp = jnp.tile(jnp.pad(gat_bias, ((0, 0), (0, 7)))[:, None, :], (1, NB, 1))
    gbp = gbp.reshape(5, NB * 16)
    wqp, wkp, wvp = _pack_fuse_w(Wq), _pack_fuse_w(Wk), _pack_fuse_w(Wv)
    wfp = jnp.pad(Wf, ((0, 0), (0, 128 - NCLS)))
    bfp = jnp.pad(bf, ((0, 128 - NCLS),))

    stats = _bn_stats(x_c)
    stats_b = jnp.transpose(stats.reshape(2, 5, 16), (1, 0, 2))

    xl_t, xr_t = _projections(x_p, stats_b, bn_w, bn_b, wl2, bl2, wr2, br2)

    part = _edge_phase(xl_t, xr_t, edges, att_r)

    pv = part.reshape(5, 2, BS, NB * 16)
    o = _fusion(pv, gbp, wqp, bq, wkp, bk, wvp, bv, wfp, bfp)
    return o[:, :NCLS]
